# Initial kernel scaffold; baseline (speedup 1.0000x reference)
import jax
import jax.numpy as jnp
from jax.experimental import pallas as pl


def kernel(boxes, scores):
    raise NotImplementedError("write your pallas kernel here")


# speedup vs baseline: 79.9158x; 79.9158x over previous
import jax
import jax.numpy as jnp
from jax.experimental import pallas as pl
from jax.experimental.pallas import tpu as pltpu

_NMS_THRESH = 0.5
_LANES = 128
_SUBLANES = 8
_BIG = 1.0e9


def _nms_kernel(n_boxes, x1_ref, y1_ref, x2_ref, y2_ref, keep_ref,
                area_ref, m_ref):
    shape = x1_ref.shape
    f32 = jnp.float32

    def pos_iota():
        return (jax.lax.broadcasted_iota(jnp.int32, shape, 0) * _LANES
                + jax.lax.broadcasted_iota(jnp.int32, shape, 1)).astype(f32)

    pos0 = pos_iota()
    valid = pos0 < f32(n_boxes)
    area_ref[...] = (x2_ref[...] - x1_ref[...]) * (y2_ref[...] - y1_ref[...])
    m_ref[...] = jnp.where(valid, pos0, _BIG)
    keep_ref[...] = jnp.where(valid, f32(1.0), f32(0.0))

    lane_iota = jax.lax.broadcasted_iota(
        jnp.int32, (1, _LANES), 1).astype(f32)

    def pick(ref, r, onehot):
        return jnp.sum(ref[pl.ds(r, 1), :] * onehot)

    def cond(next_pos):
        return next_pos < _BIG * 0.5

    def body(next_pos):
        rf = jnp.floor(next_pos * (1.0 / _LANES))
        r = rf.astype(jnp.int32)
        cf = next_pos - rf * _LANES
        onehot = jnp.where(lane_iota == cf, f32(1.0), f32(0.0))
        x1i = pick(x1_ref, r, onehot)
        y1i = pick(y1_ref, r, onehot)
        x2i = pick(x2_ref, r, onehot)
        y2i = pick(y2_ref, r, onehot)
        ai = pick(area_ref, r, onehot)

        x1 = x1_ref[...]
        y1 = y1_ref[...]
        x2 = x2_ref[...]
        y2 = y2_ref[...]
        area = area_ref[...]
        xx1 = jnp.maximum(x1i, x1)
        yy1 = jnp.maximum(y1i, y1)
        xx2 = jnp.minimum(x2i, x2)
        yy2 = jnp.minimum(y2i, y2)
        w = jnp.maximum(f32(1e-10), xx2 - xx1)
        h = jnp.maximum(f32(1e-10), yy2 - yy1)
        inter = w * h
        iou = inter / (ai + area - inter + f32(1e-14))

        pos = pos_iota()
        sup = (iou > _NMS_THRESH) & (pos > next_pos)
        keep_ref[...] = jnp.where(sup, f32(0.0), keep_ref[...])
        m = jnp.where(sup | (pos == next_pos), f32(_BIG), m_ref[...])
        m_ref[...] = m
        return jnp.min(m)

    next0 = jnp.min(m_ref[...])
    jax.lax.while_loop(cond, body, next0)


def kernel(boxes, scores):
    n = boxes.shape[0]
    pad_n = ((n + _LANES * _SUBLANES - 1)
             // (_LANES * _SUBLANES)) * (_LANES * _SUBLANES)
    rows = pad_n // _LANES

    xy1 = boxes[:, :2] - boxes[:, 2:] * 0.5
    xy2 = boxes[:, :2] + boxes[:, 2:] * 0.5
    boxes_xyxy = jnp.concatenate([xy1, xy2], axis=-1)

    order = jnp.argsort(-scores)
    b = boxes_xyxy[order]
    planes = [
        jnp.pad(b[:, k], (0, pad_n - n)).reshape(rows, _LANES)
        for k in range(4)
    ]

    keep_sorted = pl.pallas_call(
        lambda *refs: _nms_kernel(n, *refs),
        out_shape=jax.ShapeDtypeStruct((rows, _LANES), jnp.float32),
        scratch_shapes=[
            pltpu.VMEM((rows, _LANES), jnp.float32),
            pltpu.VMEM((rows, _LANES), jnp.float32),
        ],
    )(*planes)

    keep_s = keep_sorted.reshape(-1)[:n]
    keep = jnp.zeros((n,), boxes.dtype).at[order].set(keep_s)
    out = jnp.concatenate(
        [boxes_xyxy * keep[:, None], (scores * keep)[:, None]], axis=-1)
    return out
